# baseline (speedup 1.0000x reference)
import functools

import jax
import jax.numpy as jnp
from jax import lax
from jax.experimental import pallas as pl
from jax.experimental.pallas import tpu as pltpu
from jax.experimental.pallas import tpu_sc as plsc

_INTER = 512
_N = 128
_E = 8
_K = 2
_LANES = 16


def _route_body(idx_hbm, wts_hbm, w_hbm, idx_v, wts_v, w_v):
    cid = lax.axis_index("c")
    sid = lax.axis_index("s")

    @pl.when((cid == 0) & (sid == 0))
    def _():
        pltpu.sync_copy(idx_hbm, idx_v)
        pltpu.sync_copy(wts_hbm, wts_v)
        for i in range(_N * _E // _LANES):
            w_v[pl.ds(i * _LANES, _LANES)] = jnp.zeros((_LANES,), jnp.float32)
        lane = lax.iota(jnp.int32, _LANES)
        for k in range(_K):
            for j in range(_N // _LANES):
                t0 = j * _LANES
                idx16 = idx_v[pl.ds(k * _N + t0, _LANES)]
                w16 = wts_v[pl.ds(k * _N + t0, _LANES)]
                flat = (t0 + lane) * _E + idx16
                plsc.addupdate_scatter(w_v, [flat], w16)
        pltpu.sync_copy(w_v, w_hbm)


_route = functools.partial(
    pl.kernel,
    _route_body,
    out_type=jax.ShapeDtypeStruct((_N * _E,), jnp.float32),
    mesh=plsc.VectorSubcoreMesh(core_axis_name="c", subcore_axis_name="s"),
    compiler_params=pltpu.CompilerParams(needs_layout_passes=False),
    scratch_types=[
        pltpu.VMEM((_N * _K,), jnp.int32),
        pltpu.VMEM((_N * _K,), jnp.float32),
        pltpu.VMEM((_N * _E,), jnp.float32),
    ],
)()


def _moe_body(x_ref, gu_ref, dn_ref, w_ref, out_ref):
    e = pl.program_id(0)
    x = x_ref[...]
    proj = jax.lax.dot_general(
        x, gu_ref[0], (((1,), (1,)), ((), ())),
        preferred_element_type=jnp.float32)
    gate = proj[:, :_INTER]
    up = proj[:, _INTER:]
    h = gate * jax.nn.sigmoid(gate) * up
    out_e = jax.lax.dot_general(
        h, dn_ref[0], (((1,), (1,)), ((), ())),
        preferred_element_type=jnp.float32)
    lane = lax.broadcasted_iota(jnp.int32, (_N, _E), 1)
    w = jnp.sum(w_ref[...] * (lane == e).astype(jnp.float32),
                axis=1, keepdims=True)
    contrib = out_e * w

    @pl.when(e == 0)
    def _():
        out_ref[...] = contrib

    @pl.when(e != 0)
    def _():
        out_ref[...] += contrib


@jax.jit
def kernel(hidden_states, top_k_index, top_k_weights, gate_up_proj, down_proj):
    n, h = hidden_states.shape
    e = gate_up_proj.shape[0]
    i2 = gate_up_proj.shape[1]
    i = down_proj.shape[2]
    idx_flat = top_k_index.astype(jnp.int32).T.reshape(-1)
    wts_flat = top_k_weights.T.reshape(-1)
    w_dense = _route(idx_flat, wts_flat).reshape(n, e)
    out = pl.pallas_call(
        _moe_body,
        grid=(e,),
        in_specs=[
            pl.BlockSpec((n, h), lambda ei: (0, 0)),
            pl.BlockSpec((1, i2, h), lambda ei: (ei, 0, 0)),
            pl.BlockSpec((1, h, i), lambda ei: (ei, 0, 0)),
            pl.BlockSpec((n, e), lambda ei: (0, 0)),
        ],
        out_specs=pl.BlockSpec((n, h), lambda ei: (0, 0)),
        out_shape=jax.ShapeDtypeStruct((n, h), jnp.float32),
    )(hidden_states, gate_up_proj, down_proj, w_dense)
    return out.astype(hidden_states.dtype)
